# Initial kernel scaffold; baseline (speedup 1.0000x reference)
import jax
import jax.numpy as jnp
from jax.experimental import pallas as pl


def kernel(x, emb):
    raise NotImplementedError("write your pallas kernel here")


# speedup vs baseline: 2.7684x; 2.7684x over previous
import functools

import jax
import jax.numpy as jnp
from jax.experimental import pallas as pl
from jax.experimental.pallas import tpu as pltpu

_DIM = 1024


def _scale_body(emb_ref, out_ref, *, scale):
    out_ref[...] = emb_ref[...] * scale


@functools.partial(jax.jit, static_argnames=("seq_len",))
def _scaled_slice(emb, seq_len):
    scale = emb.shape[1] ** (-0.5)
    rows_per_block = 512
    grid = (seq_len // rows_per_block,)
    return pl.pallas_call(
        functools.partial(_scale_body, scale=scale),
        grid=grid,
        in_specs=[
            pl.BlockSpec((rows_per_block, emb.shape[1]), lambda i: (i, 0)),
        ],
        out_specs=pl.BlockSpec((rows_per_block, emb.shape[1]), lambda i: (i, 0)),
        out_shape=jax.ShapeDtypeStruct((seq_len, emb.shape[1]), emb.dtype),
    )(emb)


def kernel(x, emb):
    return _scaled_slice(emb, x.shape[1])
